# baseline (speedup 1.0000x reference)
import functools

import jax
import jax.numpy as jnp
from jax import lax
from jax.experimental import pallas as pl
from jax.experimental.pallas import tpu as pltpu
from jax.experimental.pallas import tpu_sc as plsc

N_BINS = 1024
N = 4_000_000
NM = 3_999_744
TAIL = N - NM
K = 2
PIECE = NM // K
NC = 2
NS = 16
NW = NC * NS
PER_W = PIECE // NW
CHUNK = PER_W // 3
NCHUNK = 3
FULL_SPAN = (CHUNK // 16) * 16
UNROLL = 8
TAIL_VREGS = TAIL // 16


def _lookup(bins_v, xx):
    u = xx * jnp.float32(256.0)
    iu = u.astype(jnp.int32)
    uf = iu.astype(jnp.float32)
    idx = iu + jnp.where(uf > u, jnp.int32(511), jnp.int32(512))
    idx = jnp.clip(idx, jnp.int32(0), jnp.int32(N_BINS - 1))
    return plsc.load_gather(bins_v, [idx])


def _piece_compute(piece_base, x_hbm, bins_hbm, out_hbm, bins_v,
                   x_v0, x_v1, y_v0, y_v1, si0, si1, so0, so1):
    c = lax.axis_index("c")
    s = lax.axis_index("s")
    wid = s * NC + c
    base = piece_base + wid * PER_W
    out_base = wid * PER_W
    pltpu.sync_copy(bins_hbm, bins_v)

    x_bufs = [x_v0, x_v1]
    y_bufs = [y_v0, y_v1]
    sin = [si0, si1]
    sout = [so0, so1]
    in_d = [None, None]
    out_d = [None, None]

    def issue_in(g):
        b = g % 2
        in_d[b] = pltpu.async_copy(
            x_hbm.at[pl.ds(base + g * CHUNK, CHUNK)], x_bufs[b], sin[b])

    issue_in(0)
    for g in range(NCHUNK):
        b = g % 2
        if g + 1 < NCHUNK:
            issue_in(g + 1)
        in_d[b].wait()
        if out_d[b] is not None:
            out_d[b].wait()
        x_v = x_bufs[b]
        y_v = y_bufs[b]

        @plsc.parallel_loop(0, FULL_SPAN, 16, unroll=UNROLL)
        def vloop(i):
            sl = pl.ds(i, 16)
            y_v[sl] = _lookup(bins_v, x_v[sl])

        out_d[b] = pltpu.async_copy(
            y_bufs[b], out_hbm.at[pl.ds(out_base + g * CHUNK, CHUNK)],
            sout[b])

    for d in out_d:
        if d is not None:
            d.wait()


def _make_last_body(piece_base):
    def _last_body(x_hbm, xt_hbm, bins_hbm, out_hbm, outt_hbm, bins_v,
                   x_v0, x_v1, y_v0, y_v1, si0, si1, so0, so1):
        _piece_compute(piece_base, x_hbm, bins_hbm, out_hbm, bins_v,
                       x_v0, x_v1, y_v0, y_v1, si0, si1, so0, so1)
        c = lax.axis_index("c")
        s = lax.axis_index("s")
        wid = s * NC + c

        @pl.when(wid == NW - 1)
        def _():
            pltpu.sync_copy(xt_hbm, x_v0.at[pl.ds(0, TAIL)])
            for i in range(TAIL_VREGS):
                sl = pl.ds(i * 16, 16)
                y_v0[sl] = _lookup(bins_v, x_v0[sl])
            pltpu.sync_copy(y_v0.at[pl.ds(0, TAIL)], outt_hbm)

    return _last_body


_scratch = [
    pltpu.VMEM((N_BINS,), jnp.float32),
    pltpu.VMEM((CHUNK,), jnp.float32),
    pltpu.VMEM((CHUNK,), jnp.float32),
    pltpu.VMEM((CHUNK,), jnp.float32),
    pltpu.VMEM((CHUNK,), jnp.float32),
    pltpu.SemaphoreType.DMA,
    pltpu.SemaphoreType.DMA,
    pltpu.SemaphoreType.DMA,
    pltpu.SemaphoreType.DMA,
]
_mesh = plsc.VectorSubcoreMesh(
    core_axis_name="c", subcore_axis_name="s", num_cores=NC, num_subcores=NS
)
_params = pltpu.CompilerParams(
    use_tc_tiling_on_sc=False, needs_layout_passes=False
)

_piece_calls = [
    functools.partial(
        pl.kernel,
        out_type=jax.ShapeDtypeStruct((PIECE,), jnp.float32),
        mesh=_mesh,
        scratch_types=_scratch,
        compiler_params=_params,
        name=f"pcn_piece{j}",
    )(functools.partial(_piece_compute, j * PIECE))
    for j in range(K - 1)
]

_last_call = functools.partial(
    pl.kernel,
    out_type=(jax.ShapeDtypeStruct((PIECE,), jnp.float32),
              jax.ShapeDtypeStruct((TAIL,), jnp.float32)),
    mesh=_mesh,
    scratch_types=_scratch,
    compiler_params=_params,
    name="pcn_last",
)(_make_last_body((K - 1) * PIECE))


@jax.jit
def kernel(x, bin_values):
    xm = x.reshape(N)[:NM]
    x_tail = x[NM:, 0]
    outs = []
    for j in range(K - 1):
        outs.append(_piece_calls[j](xm, bin_values)[:, None])
    o_last, o_tail = _last_call(xm, x_tail, bin_values)
    outs.append(o_last[:, None])
    outs.append(o_tail[:, None])
    return jnp.concatenate(outs, axis=0)


# speedup vs baseline: 1.1195x; 1.1195x over previous
import functools

import jax
import jax.numpy as jnp
from jax import lax
from jax.experimental import pallas as pl
from jax.experimental.pallas import tpu as pltpu
from jax.experimental.pallas import tpu_sc as plsc

N_BINS = 1024
N = 4_000_000
NM = 3_999_744
TAIL = N - NM
K = 9
PIECE = NM // K
NC = 2
NS = 16
NW = NC * NS
PER_W = PIECE // NW
CHUNK = PER_W
NCHUNK = 1
FULL_SPAN = (CHUNK // 16) * 16
UNROLL = 8
TAIL_VREGS = TAIL // 16


def _lookup(bins_v, xx):
    u = xx * jnp.float32(256.0)
    iu = u.astype(jnp.int32)
    uf = iu.astype(jnp.float32)
    idx = iu + jnp.where(uf > u, jnp.int32(511), jnp.int32(512))
    idx = jnp.clip(idx, jnp.int32(0), jnp.int32(N_BINS - 1))
    return plsc.load_gather(bins_v, [idx])


def _piece_compute(piece_base, x_hbm, bins_hbm, out_hbm, bins_v,
                   x_v0, x_v1, y_v0, y_v1, si0, si1, so0, so1):
    c = lax.axis_index("c")
    s = lax.axis_index("s")
    wid = s * NC + c
    base = piece_base + wid * PER_W
    out_base = wid * PER_W
    pltpu.sync_copy(bins_hbm, bins_v)

    x_bufs = [x_v0, x_v1]
    y_bufs = [y_v0, y_v1]
    sin = [si0, si1]
    sout = [so0, so1]
    in_d = [None, None]
    out_d = [None, None]

    def issue_in(g):
        b = g % 2
        in_d[b] = pltpu.async_copy(
            x_hbm.at[pl.ds(base + g * CHUNK, CHUNK)], x_bufs[b], sin[b])

    issue_in(0)
    for g in range(NCHUNK):
        b = g % 2
        if g + 1 < NCHUNK:
            issue_in(g + 1)
        in_d[b].wait()
        if out_d[b] is not None:
            out_d[b].wait()
        x_v = x_bufs[b]
        y_v = y_bufs[b]

        @plsc.parallel_loop(0, FULL_SPAN, 16, unroll=UNROLL)
        def vloop(i):
            sl = pl.ds(i, 16)
            y_v[sl] = _lookup(bins_v, x_v[sl])

        out_d[b] = pltpu.async_copy(
            y_bufs[b], out_hbm.at[pl.ds(out_base + g * CHUNK, CHUNK)],
            sout[b])

    for d in out_d:
        if d is not None:
            d.wait()


def _make_last_body(piece_base):
    def _last_body(x_hbm, xt_hbm, bins_hbm, out_hbm, outt_hbm, bins_v,
                   x_v0, x_v1, y_v0, y_v1, si0, si1, so0, so1):
        _piece_compute(piece_base, x_hbm, bins_hbm, out_hbm, bins_v,
                       x_v0, x_v1, y_v0, y_v1, si0, si1, so0, so1)
        c = lax.axis_index("c")
        s = lax.axis_index("s")
        wid = s * NC + c

        @pl.when(wid == NW - 1)
        def _():
            pltpu.sync_copy(xt_hbm, x_v0.at[pl.ds(0, TAIL)])
            for i in range(TAIL_VREGS):
                sl = pl.ds(i * 16, 16)
                y_v0[sl] = _lookup(bins_v, x_v0[sl])
            pltpu.sync_copy(y_v0.at[pl.ds(0, TAIL)], outt_hbm)

    return _last_body


_scratch = [
    pltpu.VMEM((N_BINS,), jnp.float32),
    pltpu.VMEM((CHUNK,), jnp.float32),
    pltpu.VMEM((CHUNK,), jnp.float32),
    pltpu.VMEM((CHUNK,), jnp.float32),
    pltpu.VMEM((CHUNK,), jnp.float32),
    pltpu.SemaphoreType.DMA,
    pltpu.SemaphoreType.DMA,
    pltpu.SemaphoreType.DMA,
    pltpu.SemaphoreType.DMA,
]
_mesh = plsc.VectorSubcoreMesh(
    core_axis_name="c", subcore_axis_name="s", num_cores=NC, num_subcores=NS
)
_params = pltpu.CompilerParams(
    use_tc_tiling_on_sc=False, needs_layout_passes=False
)

_piece_calls = [
    functools.partial(
        pl.kernel,
        out_type=jax.ShapeDtypeStruct((PIECE,), jnp.float32),
        mesh=_mesh,
        scratch_types=_scratch,
        compiler_params=_params,
        name=f"pcn_piece{j}",
    )(functools.partial(_piece_compute, j * PIECE))
    for j in range(K - 1)
]

_last_call = functools.partial(
    pl.kernel,
    out_type=(jax.ShapeDtypeStruct((PIECE,), jnp.float32),
              jax.ShapeDtypeStruct((TAIL,), jnp.float32)),
    mesh=_mesh,
    scratch_types=_scratch,
    compiler_params=_params,
    name="pcn_last",
)(_make_last_body((K - 1) * PIECE))


@jax.jit
def kernel(x, bin_values):
    xm = x.reshape(N)[:NM]
    x_tail = x[NM:, 0]
    outs = []
    for j in range(K - 1):
        outs.append(_piece_calls[j](xm, bin_values)[:, None])
    o_last, o_tail = _last_call(xm, x_tail, bin_values)
    outs.append(o_last[:, None])
    outs.append(o_tail[:, None])
    return jnp.concatenate(outs, axis=0)


# speedup vs baseline: 1.3027x; 1.1636x over previous
import functools

import jax
import jax.numpy as jnp
from jax import lax
from jax.experimental import pallas as pl
from jax.experimental.pallas import tpu as pltpu
from jax.experimental.pallas import tpu_sc as plsc

N_BINS = 1024
N = 4_000_000
NM = 3_999_744
TAIL = N - NM
K = 6
PIECE = NM // K
NC = 2
NS = 16
NW = NC * NS
PER_W = PIECE // NW
CHUNK = PER_W // 2
NCHUNK = 2
FULL_SPAN = (CHUNK // 16) * 16
UNROLL = 8
TAIL_VREGS = TAIL // 16


def _lookup(bins_v, xx):
    u = xx * jnp.float32(256.0)
    iu = u.astype(jnp.int32)
    uf = iu.astype(jnp.float32)
    idx = iu + jnp.where(uf > u, jnp.int32(511), jnp.int32(512))
    idx = jnp.clip(idx, jnp.int32(0), jnp.int32(N_BINS - 1))
    return plsc.load_gather(bins_v, [idx])


def _piece_compute(piece_base, x_hbm, bins_hbm, out_hbm, bins_v,
                   x_v0, x_v1, y_v0, y_v1, si0, si1, so0, so1):
    c = lax.axis_index("c")
    s = lax.axis_index("s")
    wid = s * NC + c
    base = piece_base + wid * PER_W
    out_base = wid * PER_W
    pltpu.sync_copy(bins_hbm, bins_v)

    x_bufs = [x_v0, x_v1]
    y_bufs = [y_v0, y_v1]
    sin = [si0, si1]
    sout = [so0, so1]
    in_d = [None, None]
    out_d = [None, None]

    def issue_in(g):
        b = g % 2
        in_d[b] = pltpu.async_copy(
            x_hbm.at[pl.ds(base + g * CHUNK, CHUNK)], x_bufs[b], sin[b])

    issue_in(0)
    for g in range(NCHUNK):
        b = g % 2
        if g + 1 < NCHUNK:
            issue_in(g + 1)
        in_d[b].wait()
        if out_d[b] is not None:
            out_d[b].wait()
        x_v = x_bufs[b]
        y_v = y_bufs[b]

        @plsc.parallel_loop(0, FULL_SPAN, 16, unroll=UNROLL)
        def vloop(i):
            sl = pl.ds(i, 16)
            y_v[sl] = _lookup(bins_v, x_v[sl])

        out_d[b] = pltpu.async_copy(
            y_bufs[b], out_hbm.at[pl.ds(out_base + g * CHUNK, CHUNK)],
            sout[b])

    for d in out_d:
        if d is not None:
            d.wait()


def _make_last_body(piece_base):
    def _last_body(x_hbm, xt_hbm, bins_hbm, out_hbm, outt_hbm, bins_v,
                   x_v0, x_v1, y_v0, y_v1, si0, si1, so0, so1):
        _piece_compute(piece_base, x_hbm, bins_hbm, out_hbm, bins_v,
                       x_v0, x_v1, y_v0, y_v1, si0, si1, so0, so1)
        c = lax.axis_index("c")
        s = lax.axis_index("s")
        wid = s * NC + c

        @pl.when(wid == NW - 1)
        def _():
            pltpu.sync_copy(xt_hbm, x_v0.at[pl.ds(0, TAIL)])
            for i in range(TAIL_VREGS):
                sl = pl.ds(i * 16, 16)
                y_v0[sl] = _lookup(bins_v, x_v0[sl])
            pltpu.sync_copy(y_v0.at[pl.ds(0, TAIL)], outt_hbm)

    return _last_body


_scratch = [
    pltpu.VMEM((N_BINS,), jnp.float32),
    pltpu.VMEM((CHUNK,), jnp.float32),
    pltpu.VMEM((CHUNK,), jnp.float32),
    pltpu.VMEM((CHUNK,), jnp.float32),
    pltpu.VMEM((CHUNK,), jnp.float32),
    pltpu.SemaphoreType.DMA,
    pltpu.SemaphoreType.DMA,
    pltpu.SemaphoreType.DMA,
    pltpu.SemaphoreType.DMA,
]
_mesh = plsc.VectorSubcoreMesh(
    core_axis_name="c", subcore_axis_name="s", num_cores=NC, num_subcores=NS
)
_params = pltpu.CompilerParams(
    use_tc_tiling_on_sc=False, needs_layout_passes=False
)

_piece_calls = [
    functools.partial(
        pl.kernel,
        out_type=jax.ShapeDtypeStruct((PIECE,), jnp.float32),
        mesh=_mesh,
        scratch_types=_scratch,
        compiler_params=_params,
        name=f"pcn_piece{j}",
    )(functools.partial(_piece_compute, j * PIECE))
    for j in range(K - 1)
]

_last_call = functools.partial(
    pl.kernel,
    out_type=(jax.ShapeDtypeStruct((PIECE,), jnp.float32),
              jax.ShapeDtypeStruct((TAIL,), jnp.float32)),
    mesh=_mesh,
    scratch_types=_scratch,
    compiler_params=_params,
    name="pcn_last",
)(_make_last_body((K - 1) * PIECE))


@jax.jit
def kernel(x, bin_values):
    xm = x.reshape(N)[:NM]
    x_tail = x[NM:, 0]
    outs = []
    for j in range(K - 1):
        outs.append(_piece_calls[j](xm, bin_values)[:, None])
    o_last, o_tail = _last_call(xm, x_tail, bin_values)
    outs.append(o_last[:, None])
    outs.append(o_tail[:, None])
    return jnp.concatenate(outs, axis=0)
